# baseline (speedup 1.0000x reference)
import functools

import jax
import jax.numpy as jnp
from jax import lax
from jax.experimental import pallas as pl
from jax.experimental.pallas import tpu as pltpu
from jax.experimental.pallas import tpu_sc as plsc

N_ROWS = 1000000
BLK = 16384
NB = (N_ROWS + BLK - 1) // BLK
NPAD = NB * BLK
NQ = 32
DIM = 64
K = 32

G = 128
NGRP = NPAD // G
GPB = BLK // G

GV = 31
NSG = NGRP // (16 * GV)
CAND_CAP = 2080
COMPACT_AT = 512
GL_CAP = 256
GL_USE = 64

NEG_INIT = -3.0
NEG_DEAD = -2.0
BIG_I32 = 0x7FFFFFFF


def _dist_block(xs_ref, x2_ref, mu_ref, out_ref, mx_ref):
    xs = xs_ref[...]
    mub = mu_ref[...]
    dot2 = lax.dot_general(xs, mub, (((1,), (1,)), ((), ())),
                           preferred_element_type=jnp.float32)
    x2 = x2_ref[...]
    m2 = jnp.sum(mub * mub, axis=1)[None, :]
    d2 = jnp.maximum((x2 + m2) + dot2, 0.0)
    dist = jnp.sqrt(d2)
    j = pl.program_id(0)

    def finish(d):
        d3 = d.reshape(NQ, GPB, G)
        out_ref[...] = d3
        mx_ref[...] = jnp.max(d3, axis=2)

    @pl.when(j < NB - 1)
    def _():
        finish(dist)

    @pl.when(j == NB - 1)
    def _():
        col = lax.broadcasted_iota(jnp.int32, (NQ, BLK), 1)
        finish(jnp.where(col < N_ROWS - (NB - 1) * BLK, dist, -1.0))


def _dists(x, mu):
    x2 = jnp.sum(x * x, axis=-1, keepdims=True)
    return pl.pallas_call(
        _dist_block,
        grid=(NB,),
        in_specs=[
            pl.BlockSpec((NQ, DIM), lambda j: (0, 0)),
            pl.BlockSpec((NQ, 1), lambda j: (0, 0)),
            pl.BlockSpec((BLK, DIM), lambda j: (j, 0)),
        ],
        out_specs=[
            pl.BlockSpec((NQ, GPB, G), lambda j: (0, j, 0)),
            pl.BlockSpec((NQ, GPB), lambda j: (0, j)),
        ],
        out_shape=[
            jax.ShapeDtypeStruct((NQ, NGRP, G), jnp.float32),
            jax.ShapeDtypeStruct((NQ, NGRP), jnp.float32),
        ],
    )(-2.0 * x, x2, mu)


def _lane_iota():
    return lax.broadcasted_iota(jnp.int32, (16,), 0)


def _splat_f(s):
    return jnp.full((16,), s, dtype=jnp.float32)


def _splat_i(s):
    return jnp.full((16,), s, dtype=jnp.int32)


def _compact(cand_v, cand_i, top_v, top_i, thr_ref, cnt_ref):
    cnt = cnt_ref[0]
    cand_v[pl.ds(cnt, 16)] = _splat_f(NEG_DEAD)
    cand_i[pl.ds(cnt, 16)] = _splat_i(BIG_I32)
    nv = (cnt + 15) // 16

    v0 = top_v[pl.ds(0, 16)]
    v1 = top_v[pl.ds(16, 16)]
    i0 = top_i[pl.ds(0, 16)]
    i1 = top_i[pl.ds(16, 16)]
    nv0 = _splat_f(NEG_INIT)
    nv1 = _splat_f(NEG_INIT)
    ni0 = _splat_i(BIG_I32)
    ni1 = _splat_i(BIG_I32)
    lanes = _lane_iota()

    def extract(carry):
        v0, v1, i0, i1 = carry
        m = jnp.maximum(v0, v1)

        def max_body(t, m):
            return jnp.maximum(m, cand_v[pl.ds(t * 16, 16)])
        m = lax.fori_loop(0, nv, max_body, m)
        s = jnp.max(m)
        sv = _splat_f(s)

        mi = jnp.minimum(jnp.where(v0 == sv, i0, BIG_I32),
                         jnp.where(v1 == sv, i1, BIG_I32))

        def min_body(t, mi):
            cv = cand_v[pl.ds(t * 16, 16)]
            ci = cand_i[pl.ds(t * 16, 16)]
            return jnp.minimum(mi, jnp.where(cv == sv, ci, BIG_I32))
        mi = lax.fori_loop(0, nv, min_body, mi)
        si = jnp.min(mi)
        siv = _splat_i(si)

        v0 = jnp.where((v0 == sv) & (i0 == siv), NEG_DEAD, v0)
        v1 = jnp.where((v1 == sv) & (i1 == siv), NEG_DEAD, v1)

        def kill_body(t, _):
            cv = cand_v[pl.ds(t * 16, 16)]
            ci = cand_i[pl.ds(t * 16, 16)]
            cand_v[pl.ds(t * 16, 16)] = jnp.where(
                (cv == sv) & (ci == siv), NEG_DEAD, cv)
            return 0
        lax.fori_loop(0, nv, kill_body, 0)
        return (v0, v1, i0, i1), s, si

    top_v[pl.ds(0, 16)] = nv0
    top_v[pl.ds(16, 16)] = nv1
    top_i[pl.ds(0, 16)] = ni0
    top_i[pl.ds(16, 16)] = ni1

    def round_body(j, carry):
        carry, s, si = extract(carry)
        half = (j // 16) * 16
        sel = lanes == _splat_i(j - half)
        tv = top_v[pl.ds(half, 16)]
        ti = top_i[pl.ds(half, 16)]
        top_v[pl.ds(half, 16)] = jnp.where(sel, _splat_f(s), tv)
        top_i[pl.ds(half, 16)] = jnp.where(sel, _splat_i(si), ti)

        @pl.when(j == K - 1)
        def _():
            thr_ref[...] = _splat_f(s)
        return carry

    lax.fori_loop(0, K, round_body, (v0, v1, i0, i1))
    cnt_ref[0] = jnp.int32(0)


def _sc_topk_gather(dists3, maxima, mu, alpha, kappa):
    mesh = plsc.VectorSubcoreMesh(core_axis_name="c", subcore_axis_name="s")

    @functools.partial(
        pl.kernel,
        mesh=mesh,
        out_type=[
            jax.ShapeDtypeStruct((NQ, K, DIM), jnp.float32),
            jax.ShapeDtypeStruct((NQ, K), jnp.float32),
            jax.ShapeDtypeStruct((NQ, K), jnp.float32),
        ],
        scratch_types=[
            pltpu.VMEM((NGRP,), jnp.float32),
            pltpu.VMEM((CAND_CAP,), jnp.float32),
            pltpu.VMEM((CAND_CAP,), jnp.int32),
            pltpu.VMEM((GL_USE,), jnp.int32),
            pltpu.VMEM((GL_USE, G), jnp.float32),
            pltpu.VMEM((K,), jnp.float32),
            pltpu.VMEM((K,), jnp.int32),
            pltpu.VMEM((16,), jnp.float32),
            pltpu.SMEM((1,), jnp.int32),
            pltpu.VMEM((K, DIM), jnp.float32),
            pltpu.VMEM((K,), jnp.float32),
            pltpu.VMEM((K,), jnp.float32),
            pltpu.SemaphoreType.DMA,
        ],
        compiler_params=pltpu.CompilerParams(needs_layout_passes=False),
    )
    def body(dists_hbm, mx_hbm, mu_hbm, alpha_hbm, kappa_hbm,
             out_mu, out_a, out_k,
             mx_v, cand_v, cand_i, gl_v, grp_v,
             top_v, top_i, thr_ref, cnt_ref,
             rows_v, a_v, k_v, sem):
        q = lax.axis_index("s") * 2 + lax.axis_index("c")
        lanes = _lane_iota()

        pltpu.sync_copy(mx_hbm.at[q], mx_v)

        def fill_warm(t, _):
            cand_v[pl.ds(t * 16, 16)] = mx_v[pl.ds(t * 16, 16)]
            cand_i[pl.ds(t * 16, 16)] = _splat_i(t * 16) + lanes
            return 0

        def scan_group(g, _):
            base = g * (16 * GV)
            thr_v = thr_ref[...]
            m = mx_v[pl.ds(base, 16)]
            for i in range(1, GV):
                m = jnp.maximum(m, mx_v[pl.ds(base + i * 16, 16)])

            @pl.when(jnp.any(m >= thr_v))
            def _rescan():
                cnt = cnt_ref[0]
                for i in range(GV):
                    v = mx_v[pl.ds(base + i * 16, 16)]
                    mask = v >= thr_v
                    iv = _splat_i(base + i * 16) + lanes
                    plsc.store_compressed(cand_v.at[pl.ds(cnt, 16)], v,
                                          mask=mask)
                    plsc.store_compressed(cand_i.at[pl.ds(cnt, 16)], iv,
                                          mask=mask)
                    pc = plsc.all_reduce_population_count(mask)
                    cnt = cnt + jnp.max(pc)
                cnt_ref[0] = cnt

                @pl.when(cnt >= COMPACT_AT)
                def _():
                    _compact(cand_v, cand_i, top_v, top_i, thr_ref, cnt_ref)
            return 0

        top_v[pl.ds(0, 16)] = _splat_f(NEG_INIT)
        top_v[pl.ds(16, 16)] = _splat_f(NEG_INIT)
        top_i[pl.ds(0, 16)] = _splat_i(BIG_I32)
        top_i[pl.ds(16, 16)] = _splat_i(BIG_I32)
        thr_ref[...] = _splat_f(NEG_INIT)
        lax.fori_loop(0, GV, fill_warm, 0)
        cnt_ref[0] = jnp.int32(GV * 16)
        _compact(cand_v, cand_i, top_v, top_i, thr_ref, cnt_ref)
        lax.fori_loop(1, NSG, scan_group, 0)
        _compact(cand_v, cand_i, top_v, top_i, thr_ref, cnt_ref)

        tstar = thr_ref[...]
        for rb in range(GL_USE // 16):
            gl_v[pl.ds(rb * 16, 16)] = _splat_i(NGRP - 1) + q * NGRP

        def collect_groups(g, _):
            base = g * (16 * GV)
            cnt = cnt_ref[0]
            for i in range(GV):
                v = mx_v[pl.ds(base + i * 16, 16)]
                mask = v >= tstar
                iv = _splat_i(base + i * 16) + lanes + q * NGRP
                plsc.store_compressed(
                    gl_v.at[pl.ds(jnp.minimum(cnt, GL_USE - 16), 16)], iv,
                    mask=mask)
                pc = plsc.all_reduce_population_count(mask)
                cnt = cnt + jnp.max(pc)
            cnt_ref[0] = cnt
            return 0

        cnt_ref[0] = jnp.int32(0)
        lax.fori_loop(0, NSG, collect_groups, 0)

        pltpu.async_copy(dists_hbm.at[gl_v], grp_v, sem).wait()

        cnt_ref[0] = jnp.int32(0)

        def collect_row(r, _):
            cidv = gl_v[pl.ds((r // 16) * 16, 16)]
            gidv = (jnp.take(cidv, _splat_i(r % 16)) - q * NGRP) * G
            cnt = cnt_ref[0]
            for i in range(G // 16):
                v = grp_v[r, pl.ds(i * 16, 16)]
                mask = v >= tstar
                iv = gidv + i * 16 + lanes
                plsc.store_compressed(
                    cand_v.at[pl.ds(jnp.minimum(cnt, CAND_CAP - 32), 16)],
                    v, mask=mask)
                plsc.store_compressed(
                    cand_i.at[pl.ds(jnp.minimum(cnt, CAND_CAP - 32), 16)],
                    iv, mask=mask)
                pc = plsc.all_reduce_population_count(mask)
                cnt = cnt + jnp.max(pc)
            cnt_ref[0] = cnt
            return 0

        lax.fori_loop(0, GL_USE, collect_row, 0)
        top_v[pl.ds(0, 16)] = _splat_f(NEG_INIT)
        top_v[pl.ds(16, 16)] = _splat_f(NEG_INIT)
        top_i[pl.ds(0, 16)] = _splat_i(BIG_I32)
        top_i[pl.ds(16, 16)] = _splat_i(BIG_I32)
        _compact(cand_v, cand_i, top_v, top_i, thr_ref, cnt_ref)

        i0 = top_i[pl.ds(0, 16)]
        i1 = top_i[pl.ds(16, 16)]
        copies = []
        for j in range(K):
            iv = i0 if j < 16 else i1
            sj = jnp.reshape(lax.slice(iv, (j % 16,), (j % 16 + 1,)), ())
            copies.append(pltpu.async_copy(mu_hbm.at[sj], rows_v.at[j], sem))
        for c in copies:
            c.wait()
        pltpu.async_copy(alpha_hbm.at[top_i], a_v, sem).wait()
        pltpu.async_copy(kappa_hbm.at[top_i], k_v, sem).wait()
        pltpu.sync_copy(rows_v, out_mu.at[q])
        pltpu.sync_copy(a_v, out_a.at[q])
        pltpu.sync_copy(k_v, out_k.at[q])

    return body(dists3, maxima, mu, alpha, kappa)


def kernel(x, mu, alpha, kappa, k):
    dists3, maxima = _dists(x, mu)
    dists2 = dists3.reshape(NQ * NGRP, G)
    return tuple(_sc_topk_gather(dists2, maxima, mu, alpha, kappa))


# speedup vs baseline: 1.0007x; 1.0007x over previous
import functools

import jax
import jax.numpy as jnp
from jax import lax
from jax.experimental import pallas as pl
from jax.experimental.pallas import tpu as pltpu
from jax.experimental.pallas import tpu_sc as plsc

N_ROWS = 1000000
BLK = 16384
NB = (N_ROWS + BLK - 1) // BLK
NPAD = NB * BLK
NQ = 32
DIM = 64
K = 32

G = 128
NGRP = NPAD // G
GPB = BLK // G

GV = 31
NSG = NGRP // (16 * GV)
CAND_CAP = 2080
COMPACT_AT = 512
GL_CAP = 256
GL_USE = 64

NEG_INIT = -3.0
NEG_DEAD = -2.0
BIG_I32 = 0x7FFFFFFF


def _dist_block(xs_ref, x2_ref, mu_ref, out_ref, mx_ref):
    xs = xs_ref[...]
    mub = mu_ref[...]
    dot2 = lax.dot_general(xs, mub, (((1,), (1,)), ((), ())),
                           preferred_element_type=jnp.float32)
    x2 = x2_ref[...]
    m2 = jnp.sum(mub * mub, axis=1)[None, :]
    d2 = jnp.maximum((x2 + m2) + dot2, 0.0)
    dist = jnp.sqrt(d2)
    j = pl.program_id(0)
    col = j * BLK + lax.broadcasted_iota(jnp.int32, (NQ, BLK), 1)
    dist = jnp.where(col < N_ROWS, dist, -1.0)
    d3 = dist.reshape(NQ, GPB, G)
    out_ref[...] = d3
    mx_ref[...] = jnp.max(d3, axis=2)


def _dists(x, mu):
    x2 = jnp.sum(x * x, axis=-1, keepdims=True)
    return pl.pallas_call(
        _dist_block,
        grid=(NB,),
        in_specs=[
            pl.BlockSpec((NQ, DIM), lambda j: (0, 0)),
            pl.BlockSpec((NQ, 1), lambda j: (0, 0)),
            pl.BlockSpec((BLK, DIM), lambda j: (j, 0)),
        ],
        out_specs=[
            pl.BlockSpec((NQ, GPB, G), lambda j: (0, j, 0)),
            pl.BlockSpec((NQ, GPB), lambda j: (0, j)),
        ],
        out_shape=[
            jax.ShapeDtypeStruct((NQ, NGRP, G), jnp.float32),
            jax.ShapeDtypeStruct((NQ, NGRP), jnp.float32),
        ],
    )(-2.0 * x, x2, mu)


def _lane_iota():
    return lax.broadcasted_iota(jnp.int32, (16,), 0)


def _splat_f(s):
    return jnp.full((16,), s, dtype=jnp.float32)


def _splat_i(s):
    return jnp.full((16,), s, dtype=jnp.int32)


def _compact(cand_v, cand_i, top_v, top_i, thr_ref, cnt_ref):
    cnt = cnt_ref[0]
    cand_v[pl.ds(cnt, 16)] = _splat_f(NEG_DEAD)
    cand_i[pl.ds(cnt, 16)] = _splat_i(BIG_I32)
    nv = (cnt + 15) // 16

    v0 = top_v[pl.ds(0, 16)]
    v1 = top_v[pl.ds(16, 16)]
    i0 = top_i[pl.ds(0, 16)]
    i1 = top_i[pl.ds(16, 16)]
    nv0 = _splat_f(NEG_INIT)
    nv1 = _splat_f(NEG_INIT)
    ni0 = _splat_i(BIG_I32)
    ni1 = _splat_i(BIG_I32)
    lanes = _lane_iota()

    def extract(carry):
        v0, v1, i0, i1 = carry
        m = jnp.maximum(v0, v1)

        def max_body(t, m):
            return jnp.maximum(m, cand_v[pl.ds(t * 16, 16)])
        m = lax.fori_loop(0, nv, max_body, m)
        s = jnp.max(m)
        sv = _splat_f(s)

        mi = jnp.minimum(jnp.where(v0 == sv, i0, BIG_I32),
                         jnp.where(v1 == sv, i1, BIG_I32))

        def min_body(t, mi):
            cv = cand_v[pl.ds(t * 16, 16)]
            ci = cand_i[pl.ds(t * 16, 16)]
            return jnp.minimum(mi, jnp.where(cv == sv, ci, BIG_I32))
        mi = lax.fori_loop(0, nv, min_body, mi)
        si = jnp.min(mi)
        siv = _splat_i(si)

        v0 = jnp.where((v0 == sv) & (i0 == siv), NEG_DEAD, v0)
        v1 = jnp.where((v1 == sv) & (i1 == siv), NEG_DEAD, v1)

        def kill_body(t, _):
            cv = cand_v[pl.ds(t * 16, 16)]
            ci = cand_i[pl.ds(t * 16, 16)]
            cand_v[pl.ds(t * 16, 16)] = jnp.where(
                (cv == sv) & (ci == siv), NEG_DEAD, cv)
            return 0
        lax.fori_loop(0, nv, kill_body, 0)
        return (v0, v1, i0, i1), s, si

    top_v[pl.ds(0, 16)] = nv0
    top_v[pl.ds(16, 16)] = nv1
    top_i[pl.ds(0, 16)] = ni0
    top_i[pl.ds(16, 16)] = ni1

    def round_body(j, carry):
        carry, s, si = extract(carry)
        half = (j // 16) * 16
        sel = lanes == _splat_i(j - half)
        tv = top_v[pl.ds(half, 16)]
        ti = top_i[pl.ds(half, 16)]
        top_v[pl.ds(half, 16)] = jnp.where(sel, _splat_f(s), tv)
        top_i[pl.ds(half, 16)] = jnp.where(sel, _splat_i(si), ti)

        @pl.when(j == K - 1)
        def _():
            thr_ref[...] = _splat_f(s)
        return carry

    lax.fori_loop(0, K, round_body, (v0, v1, i0, i1))
    cnt_ref[0] = jnp.int32(0)


def _sc_topk_gather(dists3, maxima, mu, alpha, kappa):
    mesh = plsc.VectorSubcoreMesh(core_axis_name="c", subcore_axis_name="s")

    @functools.partial(
        pl.kernel,
        mesh=mesh,
        out_type=[
            jax.ShapeDtypeStruct((NQ, K, DIM), jnp.float32),
            jax.ShapeDtypeStruct((NQ, K), jnp.float32),
            jax.ShapeDtypeStruct((NQ, K), jnp.float32),
        ],
        scratch_types=[
            pltpu.VMEM((NGRP,), jnp.float32),
            pltpu.VMEM((CAND_CAP,), jnp.float32),
            pltpu.VMEM((CAND_CAP,), jnp.int32),
            pltpu.VMEM((GL_USE,), jnp.int32),
            pltpu.VMEM((GL_USE, G), jnp.float32),
            pltpu.VMEM((K,), jnp.float32),
            pltpu.VMEM((K,), jnp.int32),
            pltpu.VMEM((16,), jnp.float32),
            pltpu.SMEM((1,), jnp.int32),
            pltpu.VMEM((K, DIM), jnp.float32),
            pltpu.VMEM((K,), jnp.float32),
            pltpu.VMEM((K,), jnp.float32),
            pltpu.SemaphoreType.DMA,
        ],
        compiler_params=pltpu.CompilerParams(needs_layout_passes=False),
    )
    def body(dists_hbm, mx_hbm, mu_hbm, alpha_hbm, kappa_hbm,
             out_mu, out_a, out_k,
             mx_v, cand_v, cand_i, gl_v, grp_v,
             top_v, top_i, thr_ref, cnt_ref,
             rows_v, a_v, k_v, sem):
        q = lax.axis_index("s") * 2 + lax.axis_index("c")
        lanes = _lane_iota()

        pltpu.sync_copy(mx_hbm.at[q], mx_v)

        def fill_warm(t, _):
            cand_v[pl.ds(t * 16, 16)] = mx_v[pl.ds(t * 16, 16)]
            cand_i[pl.ds(t * 16, 16)] = _splat_i(t * 16) + lanes
            return 0

        def scan_group(g, _):
            base = g * (16 * GV)
            thr_v = thr_ref[...]
            m = mx_v[pl.ds(base, 16)]
            for i in range(1, GV):
                m = jnp.maximum(m, mx_v[pl.ds(base + i * 16, 16)])

            @pl.when(jnp.any(m >= thr_v))
            def _rescan():
                cnt = cnt_ref[0]
                for i in range(GV):
                    v = mx_v[pl.ds(base + i * 16, 16)]
                    mask = v >= thr_v
                    iv = _splat_i(base + i * 16) + lanes
                    plsc.store_compressed(cand_v.at[pl.ds(cnt, 16)], v,
                                          mask=mask)
                    plsc.store_compressed(cand_i.at[pl.ds(cnt, 16)], iv,
                                          mask=mask)
                    pc = plsc.all_reduce_population_count(mask)
                    cnt = cnt + jnp.max(pc)
                cnt_ref[0] = cnt

                @pl.when(cnt >= COMPACT_AT)
                def _():
                    _compact(cand_v, cand_i, top_v, top_i, thr_ref, cnt_ref)
            return 0

        top_v[pl.ds(0, 16)] = _splat_f(NEG_INIT)
        top_v[pl.ds(16, 16)] = _splat_f(NEG_INIT)
        top_i[pl.ds(0, 16)] = _splat_i(BIG_I32)
        top_i[pl.ds(16, 16)] = _splat_i(BIG_I32)
        thr_ref[...] = _splat_f(NEG_INIT)
        lax.fori_loop(0, GV, fill_warm, 0)
        cnt_ref[0] = jnp.int32(GV * 16)
        _compact(cand_v, cand_i, top_v, top_i, thr_ref, cnt_ref)
        lax.fori_loop(1, NSG, scan_group, 0)
        _compact(cand_v, cand_i, top_v, top_i, thr_ref, cnt_ref)

        tstar = thr_ref[...]
        for rb in range(GL_USE // 16):
            gl_v[pl.ds(rb * 16, 16)] = _splat_i(NGRP - 1) + q * NGRP

        def collect_groups(g, _):
            base = g * (16 * GV)
            cnt = cnt_ref[0]
            for i in range(GV):
                v = mx_v[pl.ds(base + i * 16, 16)]
                mask = v >= tstar
                iv = _splat_i(base + i * 16) + lanes + q * NGRP
                plsc.store_compressed(
                    gl_v.at[pl.ds(jnp.minimum(cnt, GL_USE - 16), 16)], iv,
                    mask=mask)
                pc = plsc.all_reduce_population_count(mask)
                cnt = cnt + jnp.max(pc)
            cnt_ref[0] = cnt
            return 0

        cnt_ref[0] = jnp.int32(0)
        lax.fori_loop(0, NSG, collect_groups, 0)

        pltpu.async_copy(dists_hbm.at[gl_v], grp_v, sem).wait()

        cnt_ref[0] = jnp.int32(0)

        def collect_row(r, _):
            cidv = gl_v[pl.ds((r // 16) * 16, 16)]
            gidv = (jnp.take(cidv, _splat_i(r % 16)) - q * NGRP) * G
            cnt = cnt_ref[0]
            for i in range(G // 16):
                v = grp_v[r, pl.ds(i * 16, 16)]
                mask = v >= tstar
                iv = gidv + i * 16 + lanes
                plsc.store_compressed(
                    cand_v.at[pl.ds(jnp.minimum(cnt, CAND_CAP - 32), 16)],
                    v, mask=mask)
                plsc.store_compressed(
                    cand_i.at[pl.ds(jnp.minimum(cnt, CAND_CAP - 32), 16)],
                    iv, mask=mask)
                pc = plsc.all_reduce_population_count(mask)
                cnt = cnt + jnp.max(pc)
            cnt_ref[0] = cnt
            return 0

        lax.fori_loop(0, GL_USE, collect_row, 0)
        top_v[pl.ds(0, 16)] = _splat_f(NEG_INIT)
        top_v[pl.ds(16, 16)] = _splat_f(NEG_INIT)
        top_i[pl.ds(0, 16)] = _splat_i(BIG_I32)
        top_i[pl.ds(16, 16)] = _splat_i(BIG_I32)
        _compact(cand_v, cand_i, top_v, top_i, thr_ref, cnt_ref)

        i0 = top_i[pl.ds(0, 16)]
        i1 = top_i[pl.ds(16, 16)]
        copies = []
        for j in range(K):
            iv = i0 if j < 16 else i1
            sj = jnp.reshape(lax.slice(iv, (j % 16,), (j % 16 + 1,)), ())
            copies.append(pltpu.async_copy(mu_hbm.at[sj], rows_v.at[j], sem))
        for c in copies:
            c.wait()
        pltpu.async_copy(alpha_hbm.at[top_i], a_v, sem).wait()
        pltpu.async_copy(kappa_hbm.at[top_i], k_v, sem).wait()
        pltpu.sync_copy(rows_v, out_mu.at[q])
        pltpu.sync_copy(a_v, out_a.at[q])
        pltpu.sync_copy(k_v, out_k.at[q])

    return body(dists3, maxima, mu, alpha, kappa)


def kernel(x, mu, alpha, kappa, k):
    dists3, maxima = _dists(x, mu)
    dists2 = dists3.reshape(NQ * NGRP, G)
    return tuple(_sc_topk_gather(dists2, maxima, mu, alpha, kappa))


# speedup vs baseline: 1.0532x; 1.0525x over previous
import functools

import jax
import jax.numpy as jnp
from jax import lax
from jax.experimental import pallas as pl
from jax.experimental.pallas import tpu as pltpu
from jax.experimental.pallas import tpu_sc as plsc

N_ROWS = 1000000
BLK = 16384
NB = (N_ROWS + BLK - 1) // BLK
NPAD = NB * BLK
NQ = 32
DIM = 64
K = 32

G = 128
NGRP = NPAD // G
GPB = BLK // G

GV = 31
NSG = NGRP // (16 * GV)
CAND_CAP = 2080
COMPACT_AT = 512
GL_CAP = 256
GL_USE = 64

NEG_INIT = -3.0
NEG_DEAD = -2.0
BIG_I32 = 0x7FFFFFFF


def _dist_block(x_ref, mu_ref, out_ref, mx_ref):
    xb = x_ref[...]
    mub = mu_ref[...]
    dot = lax.dot_general(xb, mub, (((1,), (1,)), ((), ())),
                          preferred_element_type=jnp.float32)
    x2 = jnp.sum(xb * xb, axis=1, keepdims=True)
    m2 = jnp.sum(mub * mub, axis=1)[None, :]
    d2 = jnp.maximum(x2 + m2 - 2.0 * dot, 0.0)
    dist = jnp.sqrt(d2)
    j = pl.program_id(0)
    col = j * BLK + lax.broadcasted_iota(jnp.int32, (NQ, BLK), 1)
    dist = jnp.where(col < N_ROWS, dist, -1.0)
    d3 = dist.reshape(NQ, GPB, G)
    out_ref[...] = d3
    mx_ref[...] = jnp.max(d3, axis=2)


def _dists(x, mu):
    return pl.pallas_call(
        _dist_block,
        grid=(NB,),
        in_specs=[
            pl.BlockSpec((NQ, DIM), lambda j: (0, 0)),
            pl.BlockSpec((BLK, DIM), lambda j: (j, 0)),
        ],
        out_specs=[
            pl.BlockSpec((NQ, GPB, G), lambda j: (0, j, 0)),
            pl.BlockSpec((NQ, GPB), lambda j: (0, j)),
        ],
        out_shape=[
            jax.ShapeDtypeStruct((NQ, NGRP, G), jnp.float32),
            jax.ShapeDtypeStruct((NQ, NGRP), jnp.float32),
        ],
    )(x, mu)


def _lane_iota():
    return lax.broadcasted_iota(jnp.int32, (16,), 0)


def _splat_f(s):
    return jnp.full((16,), s, dtype=jnp.float32)


def _splat_i(s):
    return jnp.full((16,), s, dtype=jnp.int32)


def _compact(cand_v, cand_i, top_v, top_i, thr_ref, cnt_ref):
    cnt = cnt_ref[0]
    cand_v[pl.ds(cnt, 16)] = _splat_f(NEG_DEAD)
    cand_i[pl.ds(cnt, 16)] = _splat_i(BIG_I32)
    nv = (cnt + 15) // 16

    v0 = top_v[pl.ds(0, 16)]
    v1 = top_v[pl.ds(16, 16)]
    i0 = top_i[pl.ds(0, 16)]
    i1 = top_i[pl.ds(16, 16)]
    nv0 = _splat_f(NEG_INIT)
    nv1 = _splat_f(NEG_INIT)
    ni0 = _splat_i(BIG_I32)
    ni1 = _splat_i(BIG_I32)
    lanes = _lane_iota()

    def extract(carry):
        v0, v1, i0, i1 = carry
        m = jnp.maximum(v0, v1)

        def max_body(t, m):
            return jnp.maximum(m, cand_v[pl.ds(t * 16, 16)])
        m = lax.fori_loop(0, nv, max_body, m)
        s = jnp.max(m)
        sv = _splat_f(s)

        mi = jnp.minimum(jnp.where(v0 == sv, i0, BIG_I32),
                         jnp.where(v1 == sv, i1, BIG_I32))

        def min_body(t, mi):
            cv = cand_v[pl.ds(t * 16, 16)]
            ci = cand_i[pl.ds(t * 16, 16)]
            return jnp.minimum(mi, jnp.where(cv == sv, ci, BIG_I32))
        mi = lax.fori_loop(0, nv, min_body, mi)
        si = jnp.min(mi)
        siv = _splat_i(si)

        v0 = jnp.where((v0 == sv) & (i0 == siv), NEG_DEAD, v0)
        v1 = jnp.where((v1 == sv) & (i1 == siv), NEG_DEAD, v1)

        def kill_body(t, _):
            cv = cand_v[pl.ds(t * 16, 16)]
            ci = cand_i[pl.ds(t * 16, 16)]
            cand_v[pl.ds(t * 16, 16)] = jnp.where(
                (cv == sv) & (ci == siv), NEG_DEAD, cv)
            return 0
        lax.fori_loop(0, nv, kill_body, 0)
        return (v0, v1, i0, i1), s, si

    top_v[pl.ds(0, 16)] = nv0
    top_v[pl.ds(16, 16)] = nv1
    top_i[pl.ds(0, 16)] = ni0
    top_i[pl.ds(16, 16)] = ni1

    def round_body(j, carry):
        carry, s, si = extract(carry)
        half = (j // 16) * 16
        sel = lanes == _splat_i(j - half)
        tv = top_v[pl.ds(half, 16)]
        ti = top_i[pl.ds(half, 16)]
        top_v[pl.ds(half, 16)] = jnp.where(sel, _splat_f(s), tv)
        top_i[pl.ds(half, 16)] = jnp.where(sel, _splat_i(si), ti)

        @pl.when(j == K - 1)
        def _():
            thr_ref[...] = _splat_f(s)
        return carry

    lax.fori_loop(0, K, round_body, (v0, v1, i0, i1))
    cnt_ref[0] = jnp.int32(0)


def _sc_topk_gather(dists3, maxima, mu, alpha, kappa):
    mesh = plsc.VectorSubcoreMesh(core_axis_name="c", subcore_axis_name="s")

    @functools.partial(
        pl.kernel,
        mesh=mesh,
        out_type=[
            jax.ShapeDtypeStruct((NQ, K, DIM), jnp.float32),
            jax.ShapeDtypeStruct((NQ, K), jnp.float32),
            jax.ShapeDtypeStruct((NQ, K), jnp.float32),
        ],
        scratch_types=[
            pltpu.VMEM((NGRP,), jnp.float32),
            pltpu.VMEM((CAND_CAP,), jnp.float32),
            pltpu.VMEM((CAND_CAP,), jnp.int32),
            pltpu.VMEM((GL_USE,), jnp.int32),
            pltpu.VMEM((GL_USE, G), jnp.float32),
            pltpu.VMEM((K,), jnp.float32),
            pltpu.VMEM((K,), jnp.int32),
            pltpu.VMEM((16,), jnp.float32),
            pltpu.SMEM((1,), jnp.int32),
            pltpu.VMEM((K, DIM), jnp.float32),
            pltpu.VMEM((K,), jnp.float32),
            pltpu.VMEM((K,), jnp.float32),
            pltpu.SemaphoreType.DMA,
        ],
        compiler_params=pltpu.CompilerParams(needs_layout_passes=False),
    )
    def body(dists_hbm, mx_hbm, mu_hbm, alpha_hbm, kappa_hbm,
             out_mu, out_a, out_k,
             mx_v, cand_v, cand_i, gl_v, grp_v,
             top_v, top_i, thr_ref, cnt_ref,
             rows_v, a_v, k_v, sem):
        q = lax.axis_index("s") * 2 + lax.axis_index("c")
        lanes = _lane_iota()

        pltpu.sync_copy(mx_hbm.at[q], mx_v)

        def fill_warm(t, _):
            cand_v[pl.ds(t * 16, 16)] = mx_v[pl.ds(t * 16, 16)]
            cand_i[pl.ds(t * 16, 16)] = _splat_i(t * 16) + lanes
            return 0

        def scan_group(g, _):
            base = g * (16 * GV)
            thr_v = thr_ref[...]
            m = mx_v[pl.ds(base, 16)]
            for i in range(1, GV):
                m = jnp.maximum(m, mx_v[pl.ds(base + i * 16, 16)])

            @pl.when(jnp.any(m >= thr_v))
            def _rescan():
                cnt = cnt_ref[0]
                for i in range(GV):
                    v = mx_v[pl.ds(base + i * 16, 16)]
                    mask = v >= thr_v
                    iv = _splat_i(base + i * 16) + lanes
                    plsc.store_compressed(cand_v.at[pl.ds(cnt, 16)], v,
                                          mask=mask)
                    plsc.store_compressed(cand_i.at[pl.ds(cnt, 16)], iv,
                                          mask=mask)
                    pc = plsc.all_reduce_population_count(mask)
                    cnt = cnt + jnp.max(pc)
                cnt_ref[0] = cnt

                @pl.when(cnt >= COMPACT_AT)
                def _():
                    _compact(cand_v, cand_i, top_v, top_i, thr_ref, cnt_ref)
            return 0

        top_v[pl.ds(0, 16)] = _splat_f(NEG_INIT)
        top_v[pl.ds(16, 16)] = _splat_f(NEG_INIT)
        top_i[pl.ds(0, 16)] = _splat_i(BIG_I32)
        top_i[pl.ds(16, 16)] = _splat_i(BIG_I32)
        thr_ref[...] = _splat_f(NEG_INIT)
        lax.fori_loop(0, GV, fill_warm, 0)
        cnt_ref[0] = jnp.int32(GV * 16)
        _compact(cand_v, cand_i, top_v, top_i, thr_ref, cnt_ref)
        lax.fori_loop(1, NSG, scan_group, 0)
        _compact(cand_v, cand_i, top_v, top_i, thr_ref, cnt_ref)

        tstar = thr_ref[...]
        for rb in range(GL_USE // 16):
            gl_v[pl.ds(rb * 16, 16)] = _splat_i(NGRP - 1) + q * NGRP

        def collect_groups(g, _):
            base = g * (16 * GV)
            cnt = cnt_ref[0]
            for i in range(GV):
                v = mx_v[pl.ds(base + i * 16, 16)]
                mask = v >= tstar
                iv = _splat_i(base + i * 16) + lanes + q * NGRP
                plsc.store_compressed(
                    gl_v.at[pl.ds(jnp.minimum(cnt, GL_USE - 16), 16)], iv,
                    mask=mask)
                pc = plsc.all_reduce_population_count(mask)
                cnt = cnt + jnp.max(pc)
            cnt_ref[0] = cnt
            return 0

        cnt_ref[0] = jnp.int32(0)
        lax.fori_loop(0, NSG, collect_groups, 0)

        pltpu.async_copy(dists_hbm.at[gl_v], grp_v, sem).wait()

        cnt_ref[0] = jnp.int32(0)

        def collect_row(r, _):
            cidv = gl_v[pl.ds((r // 16) * 16, 16)]
            gidv = (jnp.take(cidv, _splat_i(r % 16)) - q * NGRP) * G
            cnt = cnt_ref[0]
            for i in range(G // 16):
                v = grp_v[r, pl.ds(i * 16, 16)]
                mask = v >= tstar
                iv = gidv + i * 16 + lanes
                plsc.store_compressed(
                    cand_v.at[pl.ds(jnp.minimum(cnt, CAND_CAP - 32), 16)],
                    v, mask=mask)
                plsc.store_compressed(
                    cand_i.at[pl.ds(jnp.minimum(cnt, CAND_CAP - 32), 16)],
                    iv, mask=mask)
                pc = plsc.all_reduce_population_count(mask)
                cnt = cnt + jnp.max(pc)
            cnt_ref[0] = cnt
            return 0

        lax.fori_loop(0, GL_USE, collect_row, 0)
        top_v[pl.ds(0, 16)] = _splat_f(NEG_INIT)
        top_v[pl.ds(16, 16)] = _splat_f(NEG_INIT)
        top_i[pl.ds(0, 16)] = _splat_i(BIG_I32)
        top_i[pl.ds(16, 16)] = _splat_i(BIG_I32)
        _compact(cand_v, cand_i, top_v, top_i, thr_ref, cnt_ref)

        i0 = top_i[pl.ds(0, 16)]
        i1 = top_i[pl.ds(16, 16)]
        copies = []
        for j in range(K):
            iv = i0 if j < 16 else i1
            sj = jnp.reshape(lax.slice(iv, (j % 16,), (j % 16 + 1,)), ())
            copies.append(pltpu.async_copy(mu_hbm.at[sj], rows_v.at[j], sem))
        for c in copies:
            c.wait()
        pltpu.async_copy(alpha_hbm.at[top_i], a_v, sem).wait()
        pltpu.async_copy(kappa_hbm.at[top_i], k_v, sem).wait()
        pltpu.sync_copy(rows_v, out_mu.at[q])
        pltpu.sync_copy(a_v, out_a.at[q])
        pltpu.sync_copy(k_v, out_k.at[q])

    return body(dists3, maxima, mu, alpha, kappa)


def kernel(x, mu, alpha, kappa, k):
    dists3, maxima = _dists(x, mu)
    dists2 = dists3.reshape(NQ * NGRP, G)
    return tuple(_sc_topk_gather(dists2, maxima, mu, alpha, kappa))


# speedup vs baseline: 1.0861x; 1.0312x over previous
import functools

import jax
import jax.numpy as jnp
from jax import lax
from jax.experimental import pallas as pl
from jax.experimental.pallas import tpu as pltpu
from jax.experimental.pallas import tpu_sc as plsc

N_ROWS = 1000000
BLK = 32768
NB = (N_ROWS + BLK - 1) // BLK
NPAD = NB * BLK
NQ = 32
DIM = 64
K = 32

G = 128
NGRP = NPAD // G
GPB = BLK // G

GV = 31
NSG = NGRP // (16 * GV)
CAND_CAP = 2080
COMPACT_AT = 512
GL_CAP = 256
GL_USE = 64

NEG_INIT = -3.0
NEG_DEAD = -2.0
BIG_I32 = 0x7FFFFFFF


def _dist_block(x_ref, mu_ref, out_ref, mx_ref):
    xb = x_ref[...]
    mub = mu_ref[...]
    dot = lax.dot_general(xb, mub, (((1,), (1,)), ((), ())),
                          preferred_element_type=jnp.float32)
    x2 = jnp.sum(xb * xb, axis=1, keepdims=True)
    m2 = jnp.sum(mub * mub, axis=1)[None, :]
    d2 = jnp.maximum(x2 + m2 - 2.0 * dot, 0.0)
    dist = jnp.sqrt(d2)
    j = pl.program_id(0)
    col = j * BLK + lax.broadcasted_iota(jnp.int32, (NQ, BLK), 1)
    dist = jnp.where(col < N_ROWS, dist, -1.0)
    d3 = dist.reshape(NQ, GPB, G)
    out_ref[...] = d3
    mx_ref[...] = jnp.max(d3, axis=2)


def _dists(x, mu):
    return pl.pallas_call(
        _dist_block,
        grid=(NB,),
        in_specs=[
            pl.BlockSpec((NQ, DIM), lambda j: (0, 0)),
            pl.BlockSpec((BLK, DIM), lambda j: (j, 0)),
        ],
        out_specs=[
            pl.BlockSpec((NQ, GPB, G), lambda j: (0, j, 0)),
            pl.BlockSpec((NQ, GPB), lambda j: (0, j)),
        ],
        out_shape=[
            jax.ShapeDtypeStruct((NQ, NGRP, G), jnp.float32),
            jax.ShapeDtypeStruct((NQ, NGRP), jnp.float32),
        ],
    )(x, mu)


def _lane_iota():
    return lax.broadcasted_iota(jnp.int32, (16,), 0)


def _splat_f(s):
    return jnp.full((16,), s, dtype=jnp.float32)


def _splat_i(s):
    return jnp.full((16,), s, dtype=jnp.int32)


def _compact(cand_v, cand_i, top_v, top_i, thr_ref, cnt_ref):
    cnt = cnt_ref[0]
    cand_v[pl.ds(cnt, 16)] = _splat_f(NEG_DEAD)
    cand_i[pl.ds(cnt, 16)] = _splat_i(BIG_I32)
    nv = (cnt + 15) // 16

    v0 = top_v[pl.ds(0, 16)]
    v1 = top_v[pl.ds(16, 16)]
    i0 = top_i[pl.ds(0, 16)]
    i1 = top_i[pl.ds(16, 16)]
    nv0 = _splat_f(NEG_INIT)
    nv1 = _splat_f(NEG_INIT)
    ni0 = _splat_i(BIG_I32)
    ni1 = _splat_i(BIG_I32)
    lanes = _lane_iota()

    def extract(carry):
        v0, v1, i0, i1 = carry
        m = jnp.maximum(v0, v1)

        def max_body(t, m):
            return jnp.maximum(m, cand_v[pl.ds(t * 16, 16)])
        m = lax.fori_loop(0, nv, max_body, m)
        s = jnp.max(m)
        sv = _splat_f(s)

        mi = jnp.minimum(jnp.where(v0 == sv, i0, BIG_I32),
                         jnp.where(v1 == sv, i1, BIG_I32))

        def min_body(t, mi):
            cv = cand_v[pl.ds(t * 16, 16)]
            ci = cand_i[pl.ds(t * 16, 16)]
            return jnp.minimum(mi, jnp.where(cv == sv, ci, BIG_I32))
        mi = lax.fori_loop(0, nv, min_body, mi)
        si = jnp.min(mi)
        siv = _splat_i(si)

        v0 = jnp.where((v0 == sv) & (i0 == siv), NEG_DEAD, v0)
        v1 = jnp.where((v1 == sv) & (i1 == siv), NEG_DEAD, v1)

        def kill_body(t, _):
            cv = cand_v[pl.ds(t * 16, 16)]
            ci = cand_i[pl.ds(t * 16, 16)]
            cand_v[pl.ds(t * 16, 16)] = jnp.where(
                (cv == sv) & (ci == siv), NEG_DEAD, cv)
            return 0
        lax.fori_loop(0, nv, kill_body, 0)
        return (v0, v1, i0, i1), s, si

    top_v[pl.ds(0, 16)] = nv0
    top_v[pl.ds(16, 16)] = nv1
    top_i[pl.ds(0, 16)] = ni0
    top_i[pl.ds(16, 16)] = ni1

    def round_body(j, carry):
        carry, s, si = extract(carry)
        half = (j // 16) * 16
        sel = lanes == _splat_i(j - half)
        tv = top_v[pl.ds(half, 16)]
        ti = top_i[pl.ds(half, 16)]
        top_v[pl.ds(half, 16)] = jnp.where(sel, _splat_f(s), tv)
        top_i[pl.ds(half, 16)] = jnp.where(sel, _splat_i(si), ti)

        @pl.when(j == K - 1)
        def _():
            thr_ref[...] = _splat_f(s)
        return carry

    lax.fori_loop(0, K, round_body, (v0, v1, i0, i1))
    cnt_ref[0] = jnp.int32(0)


def _sc_topk_gather(dists3, maxima, mu, alpha, kappa):
    mesh = plsc.VectorSubcoreMesh(core_axis_name="c", subcore_axis_name="s")

    @functools.partial(
        pl.kernel,
        mesh=mesh,
        out_type=[
            jax.ShapeDtypeStruct((NQ, K, DIM), jnp.float32),
            jax.ShapeDtypeStruct((NQ, K), jnp.float32),
            jax.ShapeDtypeStruct((NQ, K), jnp.float32),
        ],
        scratch_types=[
            pltpu.VMEM((NGRP,), jnp.float32),
            pltpu.VMEM((CAND_CAP,), jnp.float32),
            pltpu.VMEM((CAND_CAP,), jnp.int32),
            pltpu.VMEM((GL_USE,), jnp.int32),
            pltpu.VMEM((GL_USE, G), jnp.float32),
            pltpu.VMEM((K,), jnp.float32),
            pltpu.VMEM((K,), jnp.int32),
            pltpu.VMEM((16,), jnp.float32),
            pltpu.SMEM((1,), jnp.int32),
            pltpu.VMEM((K, DIM), jnp.float32),
            pltpu.VMEM((K,), jnp.float32),
            pltpu.VMEM((K,), jnp.float32),
            pltpu.SemaphoreType.DMA,
        ],
        compiler_params=pltpu.CompilerParams(needs_layout_passes=False),
    )
    def body(dists_hbm, mx_hbm, mu_hbm, alpha_hbm, kappa_hbm,
             out_mu, out_a, out_k,
             mx_v, cand_v, cand_i, gl_v, grp_v,
             top_v, top_i, thr_ref, cnt_ref,
             rows_v, a_v, k_v, sem):
        q = lax.axis_index("s") * 2 + lax.axis_index("c")
        lanes = _lane_iota()

        pltpu.sync_copy(mx_hbm.at[q], mx_v)

        def fill_warm(t, _):
            cand_v[pl.ds(t * 16, 16)] = mx_v[pl.ds(t * 16, 16)]
            cand_i[pl.ds(t * 16, 16)] = _splat_i(t * 16) + lanes
            return 0

        def scan_group(g, _):
            base = g * (16 * GV)
            thr_v = thr_ref[...]
            m = mx_v[pl.ds(base, 16)]
            for i in range(1, GV):
                m = jnp.maximum(m, mx_v[pl.ds(base + i * 16, 16)])

            @pl.when(jnp.any(m >= thr_v))
            def _rescan():
                cnt = cnt_ref[0]
                for i in range(GV):
                    v = mx_v[pl.ds(base + i * 16, 16)]
                    mask = v >= thr_v
                    iv = _splat_i(base + i * 16) + lanes
                    plsc.store_compressed(cand_v.at[pl.ds(cnt, 16)], v,
                                          mask=mask)
                    plsc.store_compressed(cand_i.at[pl.ds(cnt, 16)], iv,
                                          mask=mask)
                    pc = plsc.all_reduce_population_count(mask)
                    cnt = cnt + jnp.max(pc)
                cnt_ref[0] = cnt

                @pl.when(cnt >= COMPACT_AT)
                def _():
                    _compact(cand_v, cand_i, top_v, top_i, thr_ref, cnt_ref)
            return 0

        top_v[pl.ds(0, 16)] = _splat_f(NEG_INIT)
        top_v[pl.ds(16, 16)] = _splat_f(NEG_INIT)
        top_i[pl.ds(0, 16)] = _splat_i(BIG_I32)
        top_i[pl.ds(16, 16)] = _splat_i(BIG_I32)
        thr_ref[...] = _splat_f(NEG_INIT)
        lax.fori_loop(0, GV, fill_warm, 0)
        cnt_ref[0] = jnp.int32(GV * 16)
        _compact(cand_v, cand_i, top_v, top_i, thr_ref, cnt_ref)
        lax.fori_loop(1, NSG, scan_group, 0)
        _compact(cand_v, cand_i, top_v, top_i, thr_ref, cnt_ref)

        tstar = thr_ref[...]
        for rb in range(GL_USE // 16):
            gl_v[pl.ds(rb * 16, 16)] = _splat_i(NGRP - 1) + q * NGRP

        def collect_groups(g, _):
            base = g * (16 * GV)
            cnt = cnt_ref[0]
            for i in range(GV):
                v = mx_v[pl.ds(base + i * 16, 16)]
                mask = v >= tstar
                iv = _splat_i(base + i * 16) + lanes + q * NGRP
                plsc.store_compressed(
                    gl_v.at[pl.ds(jnp.minimum(cnt, GL_USE - 16), 16)], iv,
                    mask=mask)
                pc = plsc.all_reduce_population_count(mask)
                cnt = cnt + jnp.max(pc)
            cnt_ref[0] = cnt
            return 0

        cnt_ref[0] = jnp.int32(0)
        lax.fori_loop(0, NSG, collect_groups, 0)

        pltpu.async_copy(dists_hbm.at[gl_v], grp_v, sem).wait()

        cnt_ref[0] = jnp.int32(0)

        def collect_row(r, _):
            cidv = gl_v[pl.ds((r // 16) * 16, 16)]
            gidv = (jnp.take(cidv, _splat_i(r % 16)) - q * NGRP) * G
            cnt = cnt_ref[0]
            for i in range(G // 16):
                v = grp_v[r, pl.ds(i * 16, 16)]
                mask = v >= tstar
                iv = gidv + i * 16 + lanes
                plsc.store_compressed(
                    cand_v.at[pl.ds(jnp.minimum(cnt, CAND_CAP - 32), 16)],
                    v, mask=mask)
                plsc.store_compressed(
                    cand_i.at[pl.ds(jnp.minimum(cnt, CAND_CAP - 32), 16)],
                    iv, mask=mask)
                pc = plsc.all_reduce_population_count(mask)
                cnt = cnt + jnp.max(pc)
            cnt_ref[0] = cnt
            return 0

        lax.fori_loop(0, GL_USE, collect_row, 0)
        top_v[pl.ds(0, 16)] = _splat_f(NEG_INIT)
        top_v[pl.ds(16, 16)] = _splat_f(NEG_INIT)
        top_i[pl.ds(0, 16)] = _splat_i(BIG_I32)
        top_i[pl.ds(16, 16)] = _splat_i(BIG_I32)
        _compact(cand_v, cand_i, top_v, top_i, thr_ref, cnt_ref)

        i0 = top_i[pl.ds(0, 16)]
        i1 = top_i[pl.ds(16, 16)]
        copies = []
        for j in range(K):
            iv = i0 if j < 16 else i1
            sj = jnp.reshape(lax.slice(iv, (j % 16,), (j % 16 + 1,)), ())
            copies.append(pltpu.async_copy(mu_hbm.at[sj], rows_v.at[j], sem))
        for c in copies:
            c.wait()
        pltpu.async_copy(alpha_hbm.at[top_i], a_v, sem).wait()
        pltpu.async_copy(kappa_hbm.at[top_i], k_v, sem).wait()
        pltpu.sync_copy(rows_v, out_mu.at[q])
        pltpu.sync_copy(a_v, out_a.at[q])
        pltpu.sync_copy(k_v, out_k.at[q])

    return body(dists3, maxima, mu, alpha, kappa)


def kernel(x, mu, alpha, kappa, k):
    dists3, maxima = _dists(x, mu)
    dists2 = dists3.reshape(NQ * NGRP, G)
    return tuple(_sc_topk_gather(dists2, maxima, mu, alpha, kappa))
